# baseline (speedup 1.0000x reference)
import functools

import jax
import jax.numpy as jnp
from jax import lax
from jax.experimental import pallas as pl
from jax.experimental.pallas import tpu as pltpu
from jax.experimental.pallas import tpu_sc as plsc

_D = 64
_B = 16384
_T = 2 * _B
_NW = 32
_ROWS_PER_W = _T // _NW
_CHUNK = 128
_NCHUNK = _ROWS_PER_W // _CHUNK
_MARGIN = 1.0
_C_REG = 0.1


def _pre_body(nv_ref, nhat_ref):
    nv = nv_ref[...]
    nn = jnp.sum(nv * nv, axis=1, keepdims=True)
    nhat_ref[...] = nv / jnp.maximum(jnp.sqrt(nn), 1e-12)


def _normalize_nv(nv):
    return pl.pallas_call(
        _pre_body,
        out_shape=jax.ShapeDtypeStruct(nv.shape, jnp.float32),
    )(nv)


def _sc_body(h_hbm, t_hbm, r_hbm, ent_hbm, rel_hbm, nhat_hbm, out_hbm,
             hidx, tidx, ridx, hrow, trow, rrow, nrow, oacc, sem):
    wid = lax.axis_index("s") * 2 + lax.axis_index("c")
    base = wid * _ROWS_PER_W
    lane = lax.iota(jnp.int32, 16)
    zero = jnp.zeros((16,), jnp.float32)
    one = jnp.full((16,), 1, jnp.int32)

    def chunk_body(c, carry):
        off = base + c * _CHUNK
        pltpu.sync_copy(h_hbm.at[pl.ds(off, _CHUNK)], hidx)
        pltpu.sync_copy(t_hbm.at[pl.ds(off, _CHUNK)], tidx)
        pltpu.sync_copy(r_hbm.at[pl.ds(off, _CHUNK)], ridx)
        cp1 = pltpu.async_copy(ent_hbm.at[hidx], hrow, sem)
        cp2 = pltpu.async_copy(ent_hbm.at[tidx], trow, sem)
        cp3 = pltpu.async_copy(rel_hbm.at[ridx], rrow, sem)
        cp4 = pltpu.async_copy(nhat_hbm.at[ridx], nrow, sem)
        cp1.wait()
        cp2.wait()
        cp3.wait()
        cp4.wait()

        def group_body(g, gcarry):
            rows = jnp.full((16,), g * 16, jnp.int32) + lane
            jv = jnp.zeros((16,), jnp.int32)
            cacc = zero
            for _ in range(_D):
                gh = plsc.load_gather(hrow, [rows, jv])
                gt = plsc.load_gather(trow, [rows, jv])
                gn = plsc.load_gather(nrow, [rows, jv])
                cacc = cacc + (gh - gt) * gn
                jv = jv + one
            jv = jnp.zeros((16,), jnp.int32)
            ssacc = zero
            for _ in range(_D):
                gh = plsc.load_gather(hrow, [rows, jv])
                gt = plsc.load_gather(trow, [rows, jv])
                gn = plsc.load_gather(nrow, [rows, jv])
                gr = plsc.load_gather(rrow, [rows, jv])
                d = (gh - gt) + gr - cacc * gn
                ssacc = ssacc + d * d
                jv = jv + one
            oacc[pl.ds(c * _CHUNK + g * 16, 16)] = ssacc
            return gcarry

        lax.fori_loop(0, _CHUNK // 16, group_body, 0)
        return carry

    lax.fori_loop(0, _NCHUNK, chunk_body, 0)
    pltpu.sync_copy(oacc, out_hbm.at[pl.ds(base, _ROWS_PER_W)])


def _sc_scores(h_idx, t_idx, r_idx, ent, rel, nhat):
    mesh = plsc.VectorSubcoreMesh(core_axis_name="c", subcore_axis_name="s")
    fn = functools.partial(
        pl.kernel,
        out_type=jax.ShapeDtypeStruct((_T,), jnp.float32),
        mesh=mesh,
        scratch_types=[
            pltpu.VMEM((_CHUNK,), jnp.int32),
            pltpu.VMEM((_CHUNK,), jnp.int32),
            pltpu.VMEM((_CHUNK,), jnp.int32),
            pltpu.VMEM((_CHUNK, _D), jnp.float32),
            pltpu.VMEM((_CHUNK, _D), jnp.float32),
            pltpu.VMEM((_CHUNK, _D), jnp.float32),
            pltpu.VMEM((_CHUNK, _D), jnp.float32),
            pltpu.VMEM((_ROWS_PER_W,), jnp.float32),
            pltpu.SemaphoreType.DMA,
        ],
        compiler_params=pltpu.CompilerParams(
            needs_layout_passes=False, use_tc_tiling_on_sc=False),
    )(_sc_body)
    return fn(h_idx, t_idx, r_idx, ent, rel, nhat)


def _post_body(ss_ref, rel_ref, nv_ref, out_ref):
    s = jnp.sqrt(ss_ref[...])
    basic = jnp.mean(jnp.maximum(_MARGIN + s[:128, :] - s[128:, :], 0.0))
    rel = rel_ref[...]
    nv = nv_ref[...]
    rn = jnp.sqrt(jnp.sum(rel * rel, axis=1))
    wn = jnp.sqrt(jnp.sum(nv * nv, axis=1))
    cons = jnp.sum(jnp.abs(jnp.sum(rel * nv, axis=1) / (rn * wn)))
    out_ref[...] = jnp.broadcast_to(basic + _C_REG * cons, (1, 1))


def _post(ss, rel, nv):
    out = pl.pallas_call(
        _post_body,
        out_shape=jax.ShapeDtypeStruct((1, 1), jnp.float32),
    )(ss.reshape(_T // 128, 128), rel, nv)
    return out[0, 0]


def kernel(pos_h, pos_r, pos_t, neg_h, neg_r, neg_t,
           entity_embedding, relation_embedding, normal_vector):
    h_idx = jnp.concatenate([pos_h, neg_h]).astype(jnp.int32)
    t_idx = jnp.concatenate([pos_t, neg_t]).astype(jnp.int32)
    r_idx = jnp.concatenate([pos_r, neg_r]).astype(jnp.int32)
    nhat = _normalize_nv(normal_vector)
    ss = _sc_scores(h_idx, t_idx, r_idx, entity_embedding,
                    relation_embedding, nhat)
    return _post(ss, relation_embedding, normal_vector)


# speedup vs baseline: 1.4772x; 1.4772x over previous
import functools

import jax
import jax.numpy as jnp
from jax import lax
from jax.experimental import pallas as pl
from jax.experimental.pallas import tpu as pltpu
from jax.experimental.pallas import tpu_sc as plsc

_D = 64
_B = 16384
_T = 2 * _B
_NW = 32
_ROWS_PER_W = _T // _NW
_CHUNK = 128
_NCHUNK = _ROWS_PER_W // _CHUNK
_MARGIN = 1.0
_C_REG = 0.1


def _pre_body(rel_ref, nv_ref, q_ref):
    nv = nv_ref[...]
    nn = jnp.sum(nv * nv, axis=1, keepdims=True)
    nhat = nv / jnp.maximum(jnp.sqrt(nn), 1e-12)
    q_ref[...] = jnp.concatenate([rel_ref[...], nhat], axis=1)


def _make_q(rel, nv):
    return pl.pallas_call(
        _pre_body,
        out_shape=jax.ShapeDtypeStruct((rel.shape[0], 2 * _D), jnp.float32),
    )(rel, nv)


def _sc_body(h_hbm, t_hbm, r_hbm, ent_hbm, q_hbm, out_hbm,
             hidx_v, tidx_v, ridx_v, hrow, trow, qrow,
             oacc, sem, semq):
    wid = lax.axis_index("s") * 2 + lax.axis_index("c")
    base = wid * _ROWS_PER_W
    lane = lax.iota(jnp.int32, 16)
    zero = jnp.zeros((16,), jnp.float32)
    one = jnp.full((16,), 1, jnp.int32)

    pltpu.sync_copy(r_hbm.at[pl.ds(base, _ROWS_PER_W)], ridx_v)

    def chunk_body(c, carry):
        off = base + c * _CHUNK
        pltpu.sync_copy(h_hbm.at[pl.ds(off, _CHUNK)], hidx_v)
        pltpu.sync_copy(t_hbm.at[pl.ds(off, _CHUNK)], tidx_v)
        pltpu.async_copy(
            q_hbm.at[ridx_v.at[pl.ds(c * _CHUNK, _CHUNK)]], qrow, semq)
        for v in range(_CHUNK // 16):
            hvec = hidx_v[pl.ds(v * 16, 16)]
            tvec = tidx_v[pl.ds(v * 16, 16)]
            for l in range(16):
                k = v * 16 + l
                pltpu.async_copy(
                    ent_hbm.at[pl.ds(hvec[l], 1)], hrow.at[pl.ds(k, 1)], sem)
                pltpu.async_copy(
                    ent_hbm.at[pl.ds(tvec[l], 1)], trow.at[pl.ds(k, 1)], sem)
        pltpu.make_async_copy(ent_hbm.at[pl.ds(0, _CHUNK)], hrow, sem).wait()
        pltpu.make_async_copy(ent_hbm.at[pl.ds(0, _CHUNK)], trow, sem).wait()
        pltpu.make_async_copy(q_hbm.at[pl.ds(0, _CHUNK)], qrow, semq).wait()

        def group_body(g, gcarry):
            rows = jnp.full((16,), g * 16, jnp.int32) + lane
            jv = jnp.zeros((16,), jnp.int32)
            cacc = zero
            for _ in range(_D):
                gh = plsc.load_gather(hrow, [rows, jv])
                gt = plsc.load_gather(trow, [rows, jv])
                gn = plsc.load_gather(qrow, [rows, jv + _D])
                cacc = cacc + (gh - gt) * gn
                jv = jv + one
            jv = jnp.zeros((16,), jnp.int32)
            ssacc = zero
            for _ in range(_D):
                gh = plsc.load_gather(hrow, [rows, jv])
                gt = plsc.load_gather(trow, [rows, jv])
                gn = plsc.load_gather(qrow, [rows, jv + _D])
                gr = plsc.load_gather(qrow, [rows, jv])
                d = (gh - gt) + gr - cacc * gn
                ssacc = ssacc + d * d
                jv = jv + one
            oacc[pl.ds(c * _CHUNK + g * 16, 16)] = ssacc
            return gcarry

        lax.fori_loop(0, _CHUNK // 16, group_body, 0)
        return carry

    lax.fori_loop(0, _NCHUNK, chunk_body, 0)
    pltpu.sync_copy(oacc, out_hbm.at[pl.ds(base, _ROWS_PER_W)])


def _sc_scores(h_idx, t_idx, r_idx, ent, q):
    mesh = plsc.VectorSubcoreMesh(core_axis_name="c", subcore_axis_name="s")
    fn = functools.partial(
        pl.kernel,
        out_type=jax.ShapeDtypeStruct((_T,), jnp.float32),
        mesh=mesh,
        scratch_types=[
            pltpu.VMEM((_CHUNK,), jnp.int32),
            pltpu.VMEM((_CHUNK,), jnp.int32),
            pltpu.VMEM((_ROWS_PER_W,), jnp.int32),
            pltpu.VMEM((_CHUNK, _D), jnp.float32),
            pltpu.VMEM((_CHUNK, _D), jnp.float32),
            pltpu.VMEM((_CHUNK, 2 * _D), jnp.float32),
            pltpu.VMEM((_ROWS_PER_W,), jnp.float32),
            pltpu.SemaphoreType.DMA,
            pltpu.SemaphoreType.DMA,
        ],
        compiler_params=pltpu.CompilerParams(
            needs_layout_passes=False, use_tc_tiling_on_sc=True),
    )(_sc_body)
    return fn(h_idx, t_idx, r_idx, ent, q)


def _post_body(ss_ref, rel_ref, nv_ref, out_ref):
    s = jnp.sqrt(ss_ref[...])
    basic = jnp.mean(jnp.maximum(_MARGIN + s[:128, :] - s[128:, :], 0.0))
    rel = rel_ref[...]
    nv = nv_ref[...]
    rn = jnp.sqrt(jnp.sum(rel * rel, axis=1))
    wn = jnp.sqrt(jnp.sum(nv * nv, axis=1))
    cons = jnp.sum(jnp.abs(jnp.sum(rel * nv, axis=1) / (rn * wn)))
    out_ref[...] = jnp.broadcast_to(basic + _C_REG * cons, (1, 1))


def _post(ss, rel, nv):
    out = pl.pallas_call(
        _post_body,
        out_shape=jax.ShapeDtypeStruct((1, 1), jnp.float32),
    )(ss.reshape(_T // 128, 128), rel, nv)
    return out[0, 0]


def kernel(pos_h, pos_r, pos_t, neg_h, neg_r, neg_t,
           entity_embedding, relation_embedding, normal_vector):
    h_idx = jnp.concatenate([pos_h, neg_h]).astype(jnp.int32)
    t_idx = jnp.concatenate([pos_t, neg_t]).astype(jnp.int32)
    r_idx = jnp.concatenate([pos_r, neg_r]).astype(jnp.int32)
    q = _make_q(relation_embedding, normal_vector)
    ss = _sc_scores(h_idx, t_idx, r_idx, entity_embedding, q)
    return _post(ss, relation_embedding, normal_vector)


# speedup vs baseline: 1.5381x; 1.0412x over previous
import functools

import jax
import jax.numpy as jnp
from jax import lax
from jax.experimental import pallas as pl
from jax.experimental.pallas import tpu as pltpu
from jax.experimental.pallas import tpu_sc as plsc

_D = 64
_B = 16384
_T = 2 * _B
_NW = 32
_ROWS_PER_W = _T // _NW
_CHUNK = 128
_NCHUNK = _ROWS_PER_W // _CHUNK
_MARGIN = 1.0
_C_REG = 0.1


def _pre_body(rel_ref, nv_ref, q_ref):
    nv = nv_ref[...]
    nn = jnp.sum(nv * nv, axis=1, keepdims=True)
    nhat = nv / jnp.maximum(jnp.sqrt(nn), 1e-12)
    q_ref[...] = jnp.concatenate([rel_ref[...], nhat], axis=1)


def _make_q(rel, nv):
    return pl.pallas_call(
        _pre_body,
        out_shape=jax.ShapeDtypeStruct((rel.shape[0], 2 * _D), jnp.float32),
    )(rel, nv)


def _sc_body(h_hbm, t_hbm, r_hbm, ent_hbm, q_hbm, out_hbm,
             hidx_v, tidx_v, ridx_v,
             h_a, t_a, q_a, h_b, t_b, q_b, oacc,
             sem_a, sem_b, semq_a, semq_b):
    wid = lax.axis_index("s") * 2 + lax.axis_index("c")
    base = wid * _ROWS_PER_W
    lane = lax.iota(jnp.int32, 16)
    zero = jnp.zeros((16,), jnp.float32)
    one = jnp.full((16,), 1, jnp.int32)

    pltpu.sync_copy(h_hbm.at[pl.ds(base, _ROWS_PER_W)], hidx_v)
    pltpu.sync_copy(t_hbm.at[pl.ds(base, _ROWS_PER_W)], tidx_v)
    pltpu.sync_copy(r_hbm.at[pl.ds(base, _ROWS_PER_W)], ridx_v)

    def fire(c, hX, tX, qX, semX, semqX):
        pltpu.async_copy(
            q_hbm.at[ridx_v.at[pl.ds(c * _CHUNK, _CHUNK)]], qX, semqX)

        def fire_v(v, carry):
            hvec = hidx_v[pl.ds(c * _CHUNK + v * 16, 16)]
            tvec = tidx_v[pl.ds(c * _CHUNK + v * 16, 16)]
            for l in range(16):
                pltpu.async_copy(
                    ent_hbm.at[pl.ds(hvec[l], 1)],
                    hX.at[pl.ds(v * 16 + l, 1)], semX)
                pltpu.async_copy(
                    ent_hbm.at[pl.ds(tvec[l], 1)],
                    tX.at[pl.ds(v * 16 + l, 1)], semX)
            return carry

        lax.fori_loop(0, _CHUNK // 16, fire_v, 0)

    def drain(hX, tX, qX, semX, semqX):
        pltpu.make_async_copy(ent_hbm.at[pl.ds(0, _CHUNK)], hX, semX).wait()
        pltpu.make_async_copy(ent_hbm.at[pl.ds(0, _CHUNK)], tX, semX).wait()
        pltpu.make_async_copy(q_hbm.at[pl.ds(0, _CHUNK)], qX, semqX).wait()

    def compute(c, hX, tX, qX):
        def group_body(g, gcarry):
            rows = jnp.full((16,), g * 16, jnp.int32) + lane
            jv = jnp.zeros((16,), jnp.int32)
            cacc = zero
            for _ in range(_D):
                gh = plsc.load_gather(hX, [rows, jv])
                gt = plsc.load_gather(tX, [rows, jv])
                gn = plsc.load_gather(qX, [rows, jv + _D])
                cacc = cacc + (gh - gt) * gn
                jv = jv + one
            jv = jnp.zeros((16,), jnp.int32)
            ssacc = zero
            for _ in range(_D):
                gh = plsc.load_gather(hX, [rows, jv])
                gt = plsc.load_gather(tX, [rows, jv])
                gn = plsc.load_gather(qX, [rows, jv + _D])
                gr = plsc.load_gather(qX, [rows, jv])
                d = (gh - gt) + gr - cacc * gn
                ssacc = ssacc + d * d
                jv = jv + one
            oacc[pl.ds(c * _CHUNK + g * 16, 16)] = ssacc
            return gcarry

        lax.fori_loop(0, _CHUNK // 16, group_body, 0)

    fire(0, h_a, t_a, q_a, sem_a, semq_a)

    def pair_body(p, carry):
        c0 = 2 * p
        fire(c0 + 1, h_b, t_b, q_b, sem_b, semq_b)
        drain(h_a, t_a, q_a, sem_a, semq_a)
        compute(c0, h_a, t_a, q_a)

        @pl.when(p < _NCHUNK // 2 - 1)
        def _():
            fire(c0 + 2, h_a, t_a, q_a, sem_a, semq_a)

        drain(h_b, t_b, q_b, sem_b, semq_b)
        compute(c0 + 1, h_b, t_b, q_b)
        return carry

    lax.fori_loop(0, _NCHUNK // 2, pair_body, 0)
    pltpu.sync_copy(oacc, out_hbm.at[pl.ds(base, _ROWS_PER_W)])


def _sc_scores(h_idx, t_idx, r_idx, ent, q):
    mesh = plsc.VectorSubcoreMesh(core_axis_name="c", subcore_axis_name="s")
    fn = functools.partial(
        pl.kernel,
        out_type=jax.ShapeDtypeStruct((_T,), jnp.float32),
        mesh=mesh,
        scratch_types=[
            pltpu.VMEM((_ROWS_PER_W,), jnp.int32),
            pltpu.VMEM((_ROWS_PER_W,), jnp.int32),
            pltpu.VMEM((_ROWS_PER_W,), jnp.int32),
            pltpu.VMEM((_CHUNK, _D), jnp.float32),
            pltpu.VMEM((_CHUNK, _D), jnp.float32),
            pltpu.VMEM((_CHUNK, 2 * _D), jnp.float32),
            pltpu.VMEM((_CHUNK, _D), jnp.float32),
            pltpu.VMEM((_CHUNK, _D), jnp.float32),
            pltpu.VMEM((_CHUNK, 2 * _D), jnp.float32),
            pltpu.VMEM((_ROWS_PER_W,), jnp.float32),
            pltpu.SemaphoreType.DMA,
            pltpu.SemaphoreType.DMA,
            pltpu.SemaphoreType.DMA,
            pltpu.SemaphoreType.DMA,
        ],
        compiler_params=pltpu.CompilerParams(
            needs_layout_passes=False, use_tc_tiling_on_sc=True),
    )(_sc_body)
    return fn(h_idx, t_idx, r_idx, ent, q)


def _post_body(ss_ref, rel_ref, nv_ref, out_ref):
    s = jnp.sqrt(ss_ref[...])
    basic = jnp.mean(jnp.maximum(_MARGIN + s[:128, :] - s[128:, :], 0.0))
    rel = rel_ref[...]
    nv = nv_ref[...]
    rn = jnp.sqrt(jnp.sum(rel * rel, axis=1))
    wn = jnp.sqrt(jnp.sum(nv * nv, axis=1))
    cons = jnp.sum(jnp.abs(jnp.sum(rel * nv, axis=1) / (rn * wn)))
    out_ref[...] = jnp.broadcast_to(basic + _C_REG * cons, (1, 1))


def _post(ss, rel, nv):
    out = pl.pallas_call(
        _post_body,
        out_shape=jax.ShapeDtypeStruct((1, 1), jnp.float32),
    )(ss.reshape(_T // 128, 128), rel, nv)
    return out[0, 0]


def kernel(pos_h, pos_r, pos_t, neg_h, neg_r, neg_t,
           entity_embedding, relation_embedding, normal_vector):
    h_idx = jnp.concatenate([pos_h, neg_h]).astype(jnp.int32)
    t_idx = jnp.concatenate([pos_t, neg_t]).astype(jnp.int32)
    r_idx = jnp.concatenate([pos_r, neg_r]).astype(jnp.int32)
    q = _make_q(relation_embedding, normal_vector)
    ss = _sc_scores(h_idx, t_idx, r_idx, entity_embedding, q)
    return _post(ss, relation_embedding, normal_vector)
